# Initial kernel scaffold; baseline (speedup 1.0000x reference)
import functools

import jax
import jax.numpy as jnp
from jax import lax
from jax.experimental import pallas as pl
from jax.experimental.pallas import tpu as pltpu
from jax.experimental.pallas import tpu_sc as plsc

NUM_RULES = 100000
RULE_DIM = 64
BATCH = 4096
NUM_ACTIVE = 26

NC = 2
NS = 16
NW = NC * NS

TOTAL = BATCH * NUM_ACTIVE
B_PER_W = TOTAL // NW
CHUNK = 128
NCHUNK = B_PER_W // CHUNK
NSLOT = 13


@functools.partial(
    pl.kernel,
    mesh=plsc.VectorSubcoreMesh(core_axis_name="c", subcore_axis_name="s"),
    out_type=jax.ShapeDtypeStruct((TOTAL, RULE_DIM), jnp.float32),
    scratch_types=[
        pltpu.VMEM((NCHUNK, CHUNK), jnp.int32),
        pltpu.VMEM((NSLOT, CHUNK, RULE_DIM), jnp.float32),
        pltpu.SemaphoreType.DMA,
        pltpu.SemaphoreType.DMA,
    ],
)
def _gather(idx_hbm, table_hbm, out_hbm, idx_v, rows_v, sem_in, sem_out):
    wid = lax.axis_index("s") * NC + lax.axis_index("c")
    base = wid * B_PER_W
    pltpu.sync_copy(idx_hbm.at[wid], idx_v)
    for h in range(NCHUNK // NSLOT):
        gets = [
            pltpu.async_copy(
                table_hbm.at[idx_v.at[h * NSLOT + s]], rows_v.at[s], sem_in
            )
            for s in range(NSLOT)
        ]
        for c in gets:
            c.wait()
        puts = [
            pltpu.async_copy(
                rows_v.at[s],
                out_hbm.at[pl.ds(base + (h * NSLOT + s) * CHUNK, CHUNK)],
                sem_out,
            )
            for s in range(NSLOT)
        ]
        for c in puts:
            c.wait()


def kernel(rule_indices, rules):
    idx = rule_indices.astype(jnp.int32).reshape(NW, NCHUNK, CHUNK)
    out = _gather(idx, rules)
    return out.reshape(BATCH, NUM_ACTIVE, RULE_DIM)


# speedup vs baseline: 1.2141x; 1.2141x over previous
import functools

import jax
import jax.numpy as jnp
from jax import lax
from jax.experimental import pallas as pl
from jax.experimental.pallas import tpu as pltpu
from jax.experimental.pallas import tpu_sc as plsc

NUM_RULES = 100000
RULE_DIM = 64
BATCH = 4096
NUM_ACTIVE = 26

NC = 2
NS = 16
NW = NC * NS

TOTAL = BATCH * NUM_ACTIVE
B_PER_W = TOTAL // NW
CHUNK = 128
NCHUNK = B_PER_W // CHUNK
NSLOT = 13


@functools.partial(
    pl.kernel,
    mesh=plsc.VectorSubcoreMesh(core_axis_name="c", subcore_axis_name="s"),
    out_type=jax.ShapeDtypeStruct((TOTAL, RULE_DIM), jnp.float32),
    scratch_types=[
        pltpu.VMEM((NCHUNK, CHUNK), jnp.int32),
        pltpu.VMEM((NSLOT, CHUNK, RULE_DIM), jnp.float32),
        pltpu.SemaphoreType.DMA,
        pltpu.SemaphoreType.DMA,
    ],
    compiler_params=pltpu.CompilerParams(use_tc_tiling_on_sc=False),
)
def _gather(idx_hbm, table_hbm, out_hbm, idx_v, rows_v, sem_in, sem_out):
    wid = lax.axis_index("s") * NC + lax.axis_index("c")
    base = wid * B_PER_W
    pltpu.sync_copy(idx_hbm.at[wid], idx_v)
    for h in range(NCHUNK // NSLOT):
        gets = [
            pltpu.async_copy(
                table_hbm.at[idx_v.at[h * NSLOT + s]], rows_v.at[s], sem_in
            )
            for s in range(NSLOT)
        ]
        for c in gets:
            c.wait()
        puts = [
            pltpu.async_copy(
                rows_v.at[s],
                out_hbm.at[pl.ds(base + (h * NSLOT + s) * CHUNK, CHUNK)],
                sem_out,
            )
            for s in range(NSLOT)
        ]
        for c in puts:
            c.wait()


def kernel(rule_indices, rules):
    idx = rule_indices.astype(jnp.int32).reshape(NW, NCHUNK, CHUNK)
    out = _gather(idx, rules)
    return out.reshape(BATCH, NUM_ACTIVE, RULE_DIM)
